# Initial kernel scaffold; baseline (speedup 1.0000x reference)
import jax
import jax.numpy as jnp
from jax.experimental import pallas as pl


def kernel(x_attk, x_def, x_ball, ei_aa, ei_ad, ei_dd, ei_ab, ei_db, Wp_attk, bp_attk, Wp_def, bp_def, Wp_ball, bp_ball, W_gcn_aa, b_gcn_aa, W_gcn_dd, b_gcn_dd, Wl_ad, bl_ad, Wr_ad, Wl_ab, bl_ab, Wr_ab, Wl_db, bl_db, Wr_db, W_ih, W_hh, b_ih, b_hh, W_hp, b_hp):
    raise NotImplementedError("write your pallas kernel here")


# speedup vs baseline: 2.8311x; 2.8311x over previous
import functools

import jax
import jax.numpy as jnp
from jax import lax
from jax.experimental import pallas as pl
from jax.experimental.pallas import tpu as pltpu
from jax.experimental.pallas import tpu_sc as plsc

TT = 4
NN = 10000
EE = 320000
D = 128
NP = 10240
RPT = NP // 16
CHUNK = 128
NCH = 2560
EP = NCH * CHUNK
CPT = NCH // 16
DEG_W = 16


def _sc_agg_body(npass, mode, tbl_hbm, src_hbm, dst_hbm, zeros_hbm, out_hbm,
                 acc, idx_s, idx_d, rows):
    cid = lax.axis_index("c")
    sid = lax.axis_index("s")
    half = (npass + 1) // 2

    def pass_body(i, carry):
        p = 2 * i + cid

        def do_pass():
            if mode == "deg":
                r = p
                vi = jnp.int32(0)
            else:
                r = p // 4
                t = p % 4
                v = jnp.where(r == 3, 1, jnp.where(r == 4, 3, r))
                vi = v * 4 + t
            pltpu.sync_copy(zeros_hbm, acc.at[pl.ds(sid * RPT, RPT)])
            plsc.subcore_barrier()

            def chunk_body(k, c2):
                j = sid * CPT + k
                pltpu.sync_copy(src_hbm.at[r, j], idx_s)
                pltpu.sync_copy(dst_hbm.at[r, j], idx_d)
                pltpu.sync_copy(tbl_hbm.at[vi].at[idx_s], rows)
                pltpu.sync_copy(rows, acc.at[idx_d], add=True)
                return c2

            lax.fori_loop(0, CPT, chunk_body, 0)
            plsc.subcore_barrier()
            pltpu.sync_copy(acc.at[pl.ds(sid * RPT, RPT)],
                            out_hbm.at[p, pl.ds(sid * RPT, RPT)])
            plsc.subcore_barrier()

        if npass % 2 == 1:
            pl.when(p < npass)(do_pass)
        else:
            do_pass()
        return carry

    lax.fori_loop(0, half, pass_body, 0)


def _sc_agg(tbl, src, dst, npass, width, mode):
    mesh = plsc.VectorSubcoreMesh(core_axis_name="c", subcore_axis_name="s")
    zeros = jnp.zeros((RPT, width), jnp.float32)
    fn = pl.kernel(
        functools.partial(_sc_agg_body, npass, mode),
        out_type=jax.ShapeDtypeStruct((npass, NP, width), jnp.float32),
        mesh=mesh,
        compiler_params=pltpu.CompilerParams(use_tc_tiling_on_sc=False),
        scratch_types=[
            pltpu.VMEM_SHARED((NP, width), jnp.float32),
            pltpu.VMEM((CHUNK,), jnp.int32),
            pltpu.VMEM((CHUNK,), jnp.int32),
            pltpu.VMEM((CHUNK, width), jnp.float32),
        ],
    )
    return fn(tbl, src, dst, zeros)


def _pad_edges(ei):
    pad = jnp.full((2, EP - EE), NN, jnp.int32)
    e = jnp.concatenate([ei, pad], axis=1)
    return e[0].reshape(NCH, CHUNK), e[1].reshape(NCH, CHUNK)


def _gru_pool(zseq, W_ih, W_hh, b_ih, b_hh):
    h = jnp.zeros((zseq.shape[1], 128), jnp.float32)
    for t in range(zseq.shape[0]):
        gi = zseq[t] @ W_ih.T + b_ih
        gh = h @ W_hh.T + b_hh
        ir, iz, inn = jnp.split(gi, 3, axis=-1)
        hr, hz, hn = jnp.split(gh, 3, axis=-1)
        r = jax.nn.sigmoid(ir + hr)
        z = jax.nn.sigmoid(iz + hz)
        ng = jnp.tanh(inn + r * hn)
        h = (1.0 - z) * ng + z * h
    return h.mean(axis=0)


def kernel(x_attk, x_def, x_ball, ei_aa, ei_ad, ei_dd, ei_ab, ei_db,
           Wp_attk, bp_attk, Wp_def, bp_def, Wp_ball, bp_ball,
           W_gcn_aa, b_gcn_aa, W_gcn_dd, b_gcn_dd,
           Wl_ad, bl_ad, Wr_ad, Wl_ab, bl_ab, Wr_ab, Wl_db, bl_db, Wr_db,
           W_ih, W_hh, b_ih, b_hh, W_hp, b_hp):
    srcs, dsts = [], []
    for ei in (ei_aa, ei_ad, ei_dd, ei_ab, ei_db):
        s, d = _pad_edges(ei)
        srcs.append(s)
        dsts.append(d)
    SRC = jnp.stack(srcs)
    DST = jnp.stack(dsts)

    ones_tbl = jnp.ones((1, NP, DEG_W), jnp.float32)
    deg_out = _sc_agg(ones_tbl, DST, DST, 5, DEG_W, "deg")
    deg = deg_out[:, :NN, 0]

    dis_aa = jnp.where(deg[0] > 0, lax.rsqrt(jnp.maximum(deg[0], 1.0)), 0.0)
    invc_ad = 1.0 / jnp.maximum(deg[1], 1.0)
    dis_dd = jnp.where(deg[2] > 0, lax.rsqrt(jnp.maximum(deg[2], 1.0)), 0.0)
    invc_ab = 1.0 / jnp.maximum(deg[3], 1.0)
    invc_db = 1.0 / jnp.maximum(deg[4], 1.0)

    xa = jnp.einsum("tnd,hd->tnh", x_attk, Wp_attk) + bp_attk
    xd = jnp.einsum("tnd,hd->tnh", x_def, Wp_def) + bp_def
    xb = jnp.einsum("tnd,hd->tnh", x_ball, Wp_ball) + bp_ball

    tbl = jnp.stack([xa * dis_aa[None, :, None], xa,
                     xd * dis_dd[None, :, None], xd])
    tbl = jnp.concatenate(
        [tbl, jnp.zeros((4, TT, NP - NN, D), jnp.float32)], axis=2)
    tbl = tbl.reshape(16, NP, D)

    agg = _sc_agg(tbl, SRC, DST, 20, D, "feat")
    agg = agg[:, :NN, :].reshape(5, TT, NN, D)
    acc_aa, acc_ad, acc_dd, acc_ab, acc_db = (agg[i] for i in range(5))

    za = jnp.einsum("tnd,hd->tnh", acc_aa * dis_aa[None, :, None], W_gcn_aa) + b_gcn_aa
    zd = (jnp.einsum("tnd,hd->tnh", acc_ad * invc_ad[None, :, None], Wl_ad) + bl_ad
          + jnp.einsum("tnd,hd->tnh", xd, Wr_ad)
          + jnp.einsum("tnd,hd->tnh", acc_dd * dis_dd[None, :, None], W_gcn_dd) + b_gcn_dd)
    zb = (jnp.einsum("tnd,hd->tnh", acc_ab * invc_ab[None, :, None], Wl_ab) + bl_ab
          + jnp.einsum("tnd,hd->tnh", xb, Wr_ab)
          + jnp.einsum("tnd,hd->tnh", acc_db * invc_db[None, :, None], Wl_db) + bl_db
          + jnp.einsum("tnd,hd->tnh", xb, Wr_db))

    pooled = [_gru_pool(z, W_ih, W_hh, b_ih, b_hh) for z in (za, zd, zb)]
    Hcat = jnp.concatenate(pooled, axis=-1)
    return Hcat @ W_hp.T + b_hp


# speedup vs baseline: 3.7762x; 1.3338x over previous
import functools

import jax
import jax.numpy as jnp
from jax import lax
from jax.experimental import pallas as pl
from jax.experimental.pallas import tpu as pltpu
from jax.experimental.pallas import tpu_sc as plsc

TT = 4
NN = 10000
EE = 320000
D = 128
NP = 10240
RPT = NP // 16
CHUNK = 128
NCH = 2560
EP = NCH * CHUNK
CPT = NCH // 16
DEG_W = 16


NBUF = 2
BLK = 16
NBLK = CPT // BLK


def _sc_agg_body(npass, mode, tbl_hbm, src_hbm, dst_hbm, zeros_hbm, out_hbm,
                 acc, idx_s, idx_d, rows, gsem, ssem):
    cid = lax.axis_index("c")
    sid = lax.axis_index("s")
    half = (npass + 1) // 2

    def pass_body(i, carry):
        p = 2 * i + cid

        def do_pass():
            if mode == "deg":
                r = p
                vi = jnp.int32(0)
            else:
                r = p // 4
                t = p % 4
                v = jnp.where(r == 3, 1, jnp.where(r == 4, 3, r))
                vi = v * 4 + t
            pltpu.sync_copy(zeros_hbm, acc.at[pl.ds(sid * RPT, RPT)])
            plsc.subcore_barrier()

            def gath(k, q):
                return pltpu.make_async_copy(
                    tbl_hbm.at[vi].at[idx_s.at[k]], rows.at[q], gsem.at[q])

            def scat(k, q):
                return pltpu.make_async_copy(
                    rows.at[q], acc.at[idx_d.at[k]], ssem.at[q])

            def blk_body(blk, c3):
                base = sid * CPT + blk * BLK
                pltpu.sync_copy(src_hbm.at[r, pl.ds(base, BLK)], idx_s)
                pltpu.sync_copy(dst_hbm.at[r, pl.ds(base, BLK)], idx_d)

                def group_body(g, c2):
                    for q in range(NBUF):
                        k = g * NBUF + q
                        pl.when(g > 0)(
                            lambda q=q, k=k: scat(k - NBUF, q).wait())
                        gath(k, q).start()
                    for q in range(NBUF):
                        k = g * NBUF + q
                        gath(k, q).wait()
                        scat(k, q).start(add=True)
                    return c2

                lax.fori_loop(0, BLK // NBUF, group_body, 0)
                for q in range(NBUF):
                    scat(BLK - NBUF + q, q).wait()
                return c3

            lax.fori_loop(0, NBLK, blk_body, 0)
            plsc.subcore_barrier()
            pltpu.sync_copy(acc.at[pl.ds(sid * RPT, RPT)],
                            out_hbm.at[p, pl.ds(sid * RPT, RPT)])
            plsc.subcore_barrier()

        if npass % 2 == 1:
            pl.when(p < npass)(do_pass)
        else:
            do_pass()
        return carry

    lax.fori_loop(0, half, pass_body, 0)


def _sc_agg(tbl, src, dst, npass, width, mode):
    mesh = plsc.VectorSubcoreMesh(core_axis_name="c", subcore_axis_name="s")
    zeros = jnp.zeros((RPT, width), jnp.float32)
    fn = pl.kernel(
        functools.partial(_sc_agg_body, npass, mode),
        out_type=jax.ShapeDtypeStruct((npass, NP, width), jnp.float32),
        mesh=mesh,
        compiler_params=pltpu.CompilerParams(use_tc_tiling_on_sc=False),
        scratch_types=[
            pltpu.VMEM_SHARED((NP, width), jnp.float32),
            pltpu.VMEM((BLK, CHUNK), jnp.int32),
            pltpu.VMEM((BLK, CHUNK), jnp.int32),
            pltpu.VMEM((NBUF, CHUNK, width), jnp.float32),
            pltpu.SemaphoreType.DMA((NBUF,)),
            pltpu.SemaphoreType.DMA((NBUF,)),
        ],
    )
    return fn(tbl, src, dst, zeros)


def _pad_edges(ei):
    pad = jnp.full((2, EP - EE), NN, jnp.int32)
    e = jnp.concatenate([ei, pad], axis=1)
    return e[0].reshape(NCH, CHUNK), e[1].reshape(NCH, CHUNK)


def _gru_pool(zseq, W_ih, W_hh, b_ih, b_hh):
    h = jnp.zeros((zseq.shape[1], 128), jnp.float32)
    for t in range(zseq.shape[0]):
        gi = zseq[t] @ W_ih.T + b_ih
        gh = h @ W_hh.T + b_hh
        ir, iz, inn = jnp.split(gi, 3, axis=-1)
        hr, hz, hn = jnp.split(gh, 3, axis=-1)
        r = jax.nn.sigmoid(ir + hr)
        z = jax.nn.sigmoid(iz + hz)
        ng = jnp.tanh(inn + r * hn)
        h = (1.0 - z) * ng + z * h
    return h.mean(axis=0)


def kernel(x_attk, x_def, x_ball, ei_aa, ei_ad, ei_dd, ei_ab, ei_db,
           Wp_attk, bp_attk, Wp_def, bp_def, Wp_ball, bp_ball,
           W_gcn_aa, b_gcn_aa, W_gcn_dd, b_gcn_dd,
           Wl_ad, bl_ad, Wr_ad, Wl_ab, bl_ab, Wr_ab, Wl_db, bl_db, Wr_db,
           W_ih, W_hh, b_ih, b_hh, W_hp, b_hp):
    srcs, dsts = [], []
    for ei in (ei_aa, ei_ad, ei_dd, ei_ab, ei_db):
        s, d = _pad_edges(ei)
        srcs.append(s)
        dsts.append(d)
    SRC = jnp.stack(srcs)
    DST = jnp.stack(dsts)

    ones_tbl = jnp.ones((1, NP, DEG_W), jnp.float32)
    deg_out = _sc_agg(ones_tbl, DST, DST, 5, DEG_W, "deg")
    deg = deg_out[:, :NN, 0]

    dis_aa = jnp.where(deg[0] > 0, lax.rsqrt(jnp.maximum(deg[0], 1.0)), 0.0)
    invc_ad = 1.0 / jnp.maximum(deg[1], 1.0)
    dis_dd = jnp.where(deg[2] > 0, lax.rsqrt(jnp.maximum(deg[2], 1.0)), 0.0)
    invc_ab = 1.0 / jnp.maximum(deg[3], 1.0)
    invc_db = 1.0 / jnp.maximum(deg[4], 1.0)

    xa = jnp.einsum("tnd,hd->tnh", x_attk, Wp_attk) + bp_attk
    xd = jnp.einsum("tnd,hd->tnh", x_def, Wp_def) + bp_def
    xb = jnp.einsum("tnd,hd->tnh", x_ball, Wp_ball) + bp_ball

    tbl = jnp.stack([xa * dis_aa[None, :, None], xa,
                     xd * dis_dd[None, :, None], xd])
    tbl = jnp.concatenate(
        [tbl, jnp.zeros((4, TT, NP - NN, D), jnp.float32)], axis=2)
    tbl = tbl.reshape(16, NP, D)

    agg = _sc_agg(tbl, SRC, DST, 20, D, "feat")
    agg = agg[:, :NN, :].reshape(5, TT, NN, D)
    acc_aa, acc_ad, acc_dd, acc_ab, acc_db = (agg[i] for i in range(5))

    za = jnp.einsum("tnd,hd->tnh", acc_aa * dis_aa[None, :, None], W_gcn_aa) + b_gcn_aa
    zd = (jnp.einsum("tnd,hd->tnh", acc_ad * invc_ad[None, :, None], Wl_ad) + bl_ad
          + jnp.einsum("tnd,hd->tnh", xd, Wr_ad)
          + jnp.einsum("tnd,hd->tnh", acc_dd * dis_dd[None, :, None], W_gcn_dd) + b_gcn_dd)
    zb = (jnp.einsum("tnd,hd->tnh", acc_ab * invc_ab[None, :, None], Wl_ab) + bl_ab
          + jnp.einsum("tnd,hd->tnh", xb, Wr_ab)
          + jnp.einsum("tnd,hd->tnh", acc_db * invc_db[None, :, None], Wl_db) + bl_db
          + jnp.einsum("tnd,hd->tnh", xb, Wr_db))

    pooled = [_gru_pool(z, W_ih, W_hh, b_ih, b_hh) for z in (za, zd, zb)]
    Hcat = jnp.concatenate(pooled, axis=-1)
    return Hcat @ W_hp.T + b_hp


# speedup vs baseline: 4.0453x; 1.0713x over previous
import functools

import jax
import jax.numpy as jnp
from jax import lax
from jax.experimental import pallas as pl
from jax.experimental.pallas import tpu as pltpu
from jax.experimental.pallas import tpu_sc as plsc

TT = 4
NN = 10000
EE = 320000
D = 128
NP = 10240
RPT = NP // 16
CHUNK = 128
NCH = 2560
EP = NCH * CHUNK
CPT = NCH // 16
DEG_W = 16


NBUF = 2
BLK = 16
NBLK = CPT // BLK


def _sc_agg_body(npass, mode, tbl_hbm, src_hbm, dst_hbm, zeros_hbm, out_hbm,
                 acc, idx_s, idx_d, rows, gsem, ssem):
    cid = lax.axis_index("c")
    sid = lax.axis_index("s")
    half = (npass + 1) // 2

    def pass_body(i, carry):
        p = 2 * i + cid

        def do_pass():
            if mode == "deg":
                r = p
                vi = jnp.int32(0)
            else:
                r = p // 4
                t = p % 4
                v = jnp.where(r == 3, 1, jnp.where(r == 4, 3, r))
                vi = v * 4 + t
            pltpu.sync_copy(zeros_hbm, acc.at[pl.ds(sid * RPT, RPT)])
            plsc.subcore_barrier()

            def gath(k, q):
                return pltpu.make_async_copy(
                    tbl_hbm.at[vi].at[idx_s.at[k]], rows.at[q], gsem.at[q])

            def scat(k, q):
                return pltpu.make_async_copy(
                    rows.at[q], acc.at[idx_d.at[k]], ssem.at[q])

            def blk_body(blk, c3):
                base = sid * CPT + blk * BLK
                pltpu.sync_copy(src_hbm.at[r, pl.ds(base, BLK)], idx_s)
                pltpu.sync_copy(dst_hbm.at[r, pl.ds(base, BLK)], idx_d)

                def group_body(g, c2):
                    for q in range(NBUF):
                        k = g * NBUF + q
                        pl.when(g > 0)(
                            lambda q=q, k=k: scat(k - NBUF, q).wait())
                        gath(k, q).start()
                    for q in range(NBUF):
                        k = g * NBUF + q
                        gath(k, q).wait()
                        scat(k, q).start(add=True)
                    return c2

                lax.fori_loop(0, BLK // NBUF, group_body, 0)
                for q in range(NBUF):
                    scat(BLK - NBUF + q, q).wait()
                return c3

            lax.fori_loop(0, NBLK, blk_body, 0)
            plsc.subcore_barrier()
            pltpu.sync_copy(acc.at[pl.ds(sid * RPT, RPT)],
                            out_hbm.at[p, pl.ds(sid * RPT, RPT)])
            plsc.subcore_barrier()

        if npass % 2 == 1:
            pl.when(p < npass)(do_pass)
        else:
            do_pass()
        return carry

    lax.fori_loop(0, half, pass_body, 0)


def _sc_agg(tbl, src, dst, npass, width, mode):
    mesh = plsc.VectorSubcoreMesh(core_axis_name="c", subcore_axis_name="s")
    zeros = jnp.zeros((RPT, width), jnp.float32)
    fn = pl.kernel(
        functools.partial(_sc_agg_body, npass, mode),
        out_type=jax.ShapeDtypeStruct((npass, NP, width), jnp.float32),
        mesh=mesh,
        compiler_params=pltpu.CompilerParams(use_tc_tiling_on_sc=False),
        scratch_types=[
            pltpu.VMEM_SHARED((NP, width), jnp.float32),
            pltpu.VMEM((BLK, CHUNK), jnp.int32),
            pltpu.VMEM((BLK, CHUNK), jnp.int32),
            pltpu.VMEM((NBUF, CHUNK, width), jnp.float32),
            pltpu.SemaphoreType.DMA((NBUF,)),
            pltpu.SemaphoreType.DMA((NBUF,)),
        ],
    )
    return fn(tbl, src, dst, zeros)


def _pad_edges(ei):
    pad_s = jnp.zeros((EP - EE,), jnp.int32)
    pad_d = jnp.full((EP - EE,), NN, jnp.int32)
    s = jnp.concatenate([ei[0], pad_s]).reshape(NCH, CHUNK)
    d = jnp.concatenate([ei[1], pad_d]).reshape(NCH, CHUNK)
    return s, d


BN = 400
NB = NN // BN


def _norms_from_deg(degp):
    deg = degp[:, :, 0]
    dis_aa = jnp.where(deg[:, 0] > 0, lax.rsqrt(jnp.maximum(deg[:, 0], 1.0)), 0.0)
    invc_ad = 1.0 / jnp.maximum(deg[:, 1], 1.0)
    dis_dd = jnp.where(deg[:, 2] > 0, lax.rsqrt(jnp.maximum(deg[:, 2], 1.0)), 0.0)
    invc_ab = 1.0 / jnp.maximum(deg[:, 3], 1.0)
    invc_db = 1.0 / jnp.maximum(deg[:, 4], 1.0)
    return dis_aa, invc_ad, dis_dd, invc_ab, invc_db


def _prep_body(xa_ref, xd_ref, xb_ref, degp_ref,
               Wpa, bpa, Wpd, bpd, Wpb, bpb, tbl_ref, xb_out):
    dis_aa, _, dis_dd, _, _ = _norms_from_deg(degp_ref[...])
    for t in range(TT):
        xa = xa_ref[t] @ Wpa[...].T + bpa[...]
        xd = xd_ref[t] @ Wpd[...].T + bpd[...]
        xb = xb_ref[t] @ Wpb[...].T + bpb[...]
        tbl_ref[t] = xa * dis_aa[:, None]
        tbl_ref[4 + t] = xa
        tbl_ref[8 + t] = xd * dis_dd[:, None]
        tbl_ref[12 + t] = xd
        xb_out[t] = xb


def _tc_prep(x_attk, x_def, x_ball, degp,
             Wp_attk, bp_attk, Wp_def, bp_def, Wp_ball, bp_ball):
    return pl.pallas_call(
        _prep_body,
        grid=(NB,),
        in_specs=[
            pl.BlockSpec((TT, BN, D), lambda i: (0, i, 0)),
            pl.BlockSpec((TT, BN, D), lambda i: (0, i, 0)),
            pl.BlockSpec((TT, BN, D), lambda i: (0, i, 0)),
            pl.BlockSpec((BN, 5, 16), lambda i: (i, 0, 0)),
            pl.BlockSpec((D, D), lambda i: (0, 0)),
            pl.BlockSpec((D,), lambda i: (0,)),
            pl.BlockSpec((D, D), lambda i: (0, 0)),
            pl.BlockSpec((D,), lambda i: (0,)),
            pl.BlockSpec((D, D), lambda i: (0, 0)),
            pl.BlockSpec((D,), lambda i: (0,)),
        ],
        out_specs=[
            pl.BlockSpec((16, BN, D), lambda i: (0, i, 0)),
            pl.BlockSpec((TT, BN, D), lambda i: (0, i, 0)),
        ],
        out_shape=[
            jax.ShapeDtypeStruct((16, NP, D), jnp.float32),
            jax.ShapeDtypeStruct((TT, NN, D), jnp.float32),
        ],
    )(x_attk, x_def, x_ball, degp,
      Wp_attk, bp_attk, Wp_def, bp_def, Wp_ball, bp_ball)


def _post_body(agg_ref, tbl_ref, xb_ref, degp_ref,
               Waa, baa, Wdd, bdd,
               Wlad, blad, Wrad, Wlab, blab, Wrab, Wldb, bldb, Wrdb,
               Wih, Whh, bih, bhh, Whp, bhp, out_ref, sums):
    i = pl.program_id(0)

    @pl.when(i == 0)
    def _():
        sums[...] = jnp.zeros_like(sums)

    dis_aa, invc_ad, dis_dd, invc_ab, invc_db = _norms_from_deg(degp_ref[...])
    za, zd, zb = [], [], []
    for t in range(TT):
        za.append((agg_ref[t] * dis_aa[:, None]) @ Waa[...].T + baa[...])
        zd.append((agg_ref[4 + t] * invc_ad[:, None]) @ Wlad[...].T + blad[...]
                  + tbl_ref[12 + t] @ Wrad[...].T
                  + (agg_ref[8 + t] * dis_dd[:, None]) @ Wdd[...].T + bdd[...])
        zb.append((agg_ref[12 + t] * invc_ab[:, None]) @ Wlab[...].T + blab[...]
                  + xb_ref[t] @ Wrab[...].T
                  + (agg_ref[16 + t] * invc_db[:, None]) @ Wldb[...].T + bldb[...]
                  + xb_ref[t] @ Wrdb[...].T)

    hs = []
    for zseq in (za, zd, zb):
        h = jnp.zeros((BN, D), jnp.float32)
        for t in range(TT):
            gi = zseq[t] @ Wih[...].T + bih[...]
            gh = h @ Whh[...].T + bhh[...]
            ir, iz, inn = gi[:, :D], gi[:, D:2 * D], gi[:, 2 * D:]
            hr, hz, hn = gh[:, :D], gh[:, D:2 * D], gh[:, 2 * D:]
            rg = jax.nn.sigmoid(ir + hr)
            zg = jax.nn.sigmoid(iz + hz)
            ng = jnp.tanh(inn + rg * hn)
            h = (1.0 - zg) * ng + zg * h
        hs.append(jnp.sum(h, axis=0))

    sums[...] = sums[...] + jnp.stack(hs, axis=0)

    @pl.when(i == NB - 1)
    def _():
        hcat = (sums[...] / float(NN)).reshape(1, 3 * D)
        out_ref[...] = (hcat @ Whp[...].T + bhp[...][None, :])[0]


def _tc_post(agg, tbl, xb, degp,
             W_gcn_aa, b_gcn_aa, W_gcn_dd, b_gcn_dd,
             Wl_ad, bl_ad, Wr_ad, Wl_ab, bl_ab, Wr_ab, Wl_db, bl_db, Wr_db,
             W_ih, W_hh, b_ih, b_hh, W_hp, b_hp):
    mat = lambda shape: pl.BlockSpec(shape, lambda i: (0,) * len(shape))
    return pl.pallas_call(
        _post_body,
        grid=(NB,),
        in_specs=[
            pl.BlockSpec((20, BN, D), lambda i: (0, i, 0)),
            pl.BlockSpec((16, BN, D), lambda i: (0, i, 0)),
            pl.BlockSpec((TT, BN, D), lambda i: (0, i, 0)),
            pl.BlockSpec((BN, 5, 16), lambda i: (i, 0, 0)),
            mat((D, D)), mat((D,)), mat((D, D)), mat((D,)),
            mat((D, D)), mat((D,)), mat((D, D)),
            mat((D, D)), mat((D,)), mat((D, D)),
            mat((D, D)), mat((D,)), mat((D, D)),
            mat((3 * D, D)), mat((3 * D, D)), mat((3 * D,)), mat((3 * D,)),
            mat((D, 3 * D)), mat((D,)),
        ],
        out_specs=pl.BlockSpec((D,), lambda i: (0,)),
        out_shape=jax.ShapeDtypeStruct((D,), jnp.float32),
        scratch_shapes=[pltpu.VMEM((3, D), jnp.float32)],
    )(agg, tbl, xb, degp,
      W_gcn_aa, b_gcn_aa, W_gcn_dd, b_gcn_dd,
      Wl_ad, bl_ad, Wr_ad, Wl_ab, bl_ab, Wr_ab, Wl_db, bl_db, Wr_db,
      W_ih, W_hh, b_ih, b_hh, W_hp, b_hp)


def kernel(x_attk, x_def, x_ball, ei_aa, ei_ad, ei_dd, ei_ab, ei_db,
           Wp_attk, bp_attk, Wp_def, bp_def, Wp_ball, bp_ball,
           W_gcn_aa, b_gcn_aa, W_gcn_dd, b_gcn_dd,
           Wl_ad, bl_ad, Wr_ad, Wl_ab, bl_ab, Wr_ab, Wl_db, bl_db, Wr_db,
           W_ih, W_hh, b_ih, b_hh, W_hp, b_hp):
    srcs, dsts = [], []
    for ei in (ei_aa, ei_ad, ei_dd, ei_ab, ei_db):
        s, d = _pad_edges(ei)
        srcs.append(s)
        dsts.append(d)
    SRC = jnp.stack(srcs)
    DST = jnp.stack(dsts)

    ones_tbl = jnp.ones((1, NP, DEG_W), jnp.float32)
    deg_out = _sc_agg(ones_tbl, DST, DST, 5, DEG_W, "deg")
    degp = jnp.transpose(deg_out, (1, 0, 2))

    tbl, xb = _tc_prep(x_attk, x_def, x_ball, degp,
                       Wp_attk, bp_attk, Wp_def, bp_def, Wp_ball, bp_ball)

    agg = _sc_agg(tbl, SRC, DST, 20, D, "feat")

    return _tc_post(agg, tbl, xb, degp,
                    W_gcn_aa, b_gcn_aa, W_gcn_dd, b_gcn_dd,
                    Wl_ad, bl_ad, Wr_ad, Wl_ab, bl_ab, Wr_ab,
                    Wl_db, bl_db, Wr_db,
                    W_ih, W_hh, b_ih, b_hh, W_hp, b_hp)


# speedup vs baseline: 6.1611x; 1.5230x over previous
import functools

import jax
import jax.numpy as jnp
from jax import lax
from jax.experimental import pallas as pl
from jax.experimental.pallas import tpu as pltpu
from jax.experimental.pallas import tpu_sc as plsc

TT = 4
NN = 10000
EE = 320000
D = 128
NP = 10240
RPT = NP // 16
CHUNK = 128
NCH = 2560
EP = NCH * CHUNK
CPT = NCH // 16
DEG_W = 16


NBUF = 2
BLK = 16
NBLK = CPT // BLK
HW = 64
FBUF = 4


def _sc_agg_body(npass, mode, tbl_hbm, src_hbm, dst_hbm, zeros_hbm, out_hbm,
                 acc, idx_s, idx_d, rows, gsem, ssem):
    cid = lax.axis_index("c")
    sid = lax.axis_index("s")
    half = (npass + 1) // 2

    def pass_body(i, carry):
        p = 2 * i + cid

        def do_pass():
            if mode == "deg":
                r = p
                vi = jnp.int32(0)
            else:
                r = p // 4
                t = p % 4
                v = jnp.where(r == 3, 1, jnp.where(r == 4, 3, r))
                vi = v * 4 + t
            pltpu.sync_copy(zeros_hbm, acc.at[pl.ds(sid * RPT, RPT)])
            plsc.subcore_barrier()

            def gath(k, q):
                return pltpu.make_async_copy(
                    tbl_hbm.at[vi].at[idx_s.at[k]], rows.at[q], gsem.at[q])

            def scat(k, q):
                return pltpu.make_async_copy(
                    rows.at[q], acc.at[idx_d.at[k]], ssem.at[q])

            def blk_body(blk, c3):
                base = sid * CPT + blk * BLK
                pltpu.sync_copy(src_hbm.at[r, pl.ds(base, BLK)], idx_s)
                pltpu.sync_copy(dst_hbm.at[r, pl.ds(base, BLK)], idx_d)

                def group_body(g, c2):
                    for q in range(NBUF):
                        k = g * NBUF + q
                        pl.when(g > 0)(
                            lambda q=q, k=k: scat(k - NBUF, q).wait())
                        gath(k, q).start()
                    for q in range(NBUF):
                        k = g * NBUF + q
                        gath(k, q).wait()
                        scat(k, q).start(add=True)
                    return c2

                lax.fori_loop(0, BLK // NBUF, group_body, 0)
                for q in range(NBUF):
                    scat(BLK - NBUF + q, q).wait()
                return c3

            lax.fori_loop(0, NBLK, blk_body, 0)
            plsc.subcore_barrier()
            pltpu.sync_copy(acc.at[pl.ds(sid * RPT, RPT)],
                            out_hbm.at[p, pl.ds(sid * RPT, RPT)])
            plsc.subcore_barrier()

        if npass % 2 == 1:
            pl.when(p < npass)(do_pass)
        else:
            do_pass()
        return carry

    lax.fori_loop(0, half, pass_body, 0)


def _sc_agg(tbl, src, dst, npass, width, mode):
    mesh = plsc.VectorSubcoreMesh(core_axis_name="c", subcore_axis_name="s")
    zeros = jnp.zeros((RPT, width), jnp.float32)
    fn = pl.kernel(
        functools.partial(_sc_agg_body, npass, mode),
        out_type=jax.ShapeDtypeStruct((npass, NP, width), jnp.float32),
        mesh=mesh,
        compiler_params=pltpu.CompilerParams(use_tc_tiling_on_sc=False),
        scratch_types=[
            pltpu.VMEM_SHARED((NP, width), jnp.float32),
            pltpu.VMEM((BLK, CHUNK), jnp.int32),
            pltpu.VMEM((BLK, CHUNK), jnp.int32),
            pltpu.VMEM((NBUF, CHUNK, width), jnp.float32),
            pltpu.SemaphoreType.DMA((NBUF,)),
            pltpu.SemaphoreType.DMA((NBUF,)),
        ],
    )
    return fn(tbl, src, dst, zeros)


def _sc_feat_body(tbl_hbm, src_hbm, dst_hbm, zeros_hbm, out_hbm,
                  tbl_sp, acc, idx_s, idx_d, rows, gsem, ssem):
    cid = lax.axis_index("c")
    sid = lax.axis_index("s")

    def pass_body(i, carry):
        r = i // 4
        t = i % 4
        v = jnp.where(r == 3, 1, jnp.where(r == 4, 3, r))
        vi = v * 4 + t
        stripe = pl.ds(sid * RPT, RPT)
        pltpu.sync_copy(tbl_hbm.at[vi, cid, stripe], tbl_sp.at[stripe])
        pltpu.sync_copy(zeros_hbm, acc.at[stripe])
        plsc.subcore_barrier()

        def gath(k, q):
            return pltpu.make_async_copy(
                tbl_sp.at[idx_s.at[k]], rows.at[q], gsem.at[q])

        def scat(k, q):
            return pltpu.make_async_copy(
                rows.at[q], acc.at[idx_d.at[k]], ssem.at[q])

        def blk_body(blk, c3):
            base = sid * CPT + blk * BLK
            pltpu.sync_copy(src_hbm.at[r, pl.ds(base, BLK)], idx_s)
            pltpu.sync_copy(dst_hbm.at[r, pl.ds(base, BLK)], idx_d)

            def group_body(g, c2):
                for q in range(FBUF):
                    k = g * FBUF + q
                    pl.when(g > 0)(lambda q=q, k=k: scat(k - FBUF, q).wait())
                    gath(k, q).start()
                for q in range(FBUF):
                    k = g * FBUF + q
                    gath(k, q).wait()
                    scat(k, q).start(add=True)
                return c2

            lax.fori_loop(0, BLK // FBUF, group_body, 0)
            for q in range(FBUF):
                scat(BLK - FBUF + q, q).wait()
            return c3

        lax.fori_loop(0, NBLK, blk_body, 0)
        plsc.subcore_barrier()
        pltpu.sync_copy(acc.at[stripe], out_hbm.at[i, cid, stripe])
        plsc.subcore_barrier()
        return carry

    lax.fori_loop(0, 20, pass_body, 0)


def _sc_feat(tbl, src, dst):
    mesh = plsc.VectorSubcoreMesh(core_axis_name="c", subcore_axis_name="s")
    zeros = jnp.zeros((RPT, HW), jnp.float32)
    fn = pl.kernel(
        _sc_feat_body,
        out_type=jax.ShapeDtypeStruct((20, 2, NP, HW), jnp.float32),
        mesh=mesh,
        compiler_params=pltpu.CompilerParams(use_tc_tiling_on_sc=False),
        scratch_types=[
            pltpu.VMEM_SHARED((NP, HW), jnp.float32),
            pltpu.VMEM_SHARED((NP, HW), jnp.float32),
            pltpu.VMEM((BLK, CHUNK), jnp.int32),
            pltpu.VMEM((BLK, CHUNK), jnp.int32),
            pltpu.VMEM((FBUF, CHUNK, HW), jnp.float32),
            pltpu.SemaphoreType.DMA((FBUF,)),
            pltpu.SemaphoreType.DMA((FBUF,)),
        ],
    )
    return fn(tbl, src, dst, zeros)


def _pad_edges(ei):
    pad_s = jnp.zeros((EP - EE,), jnp.int32)
    pad_d = jnp.full((EP - EE,), NN, jnp.int32)
    s = jnp.concatenate([ei[0], pad_s]).reshape(NCH, CHUNK)
    d = jnp.concatenate([ei[1], pad_d]).reshape(NCH, CHUNK)
    return s, d


BN = 400
NB = NN // BN


def _norms_from_deg(degp):
    deg = degp[:, :, 0]
    dis_aa = jnp.where(deg[:, 0] > 0, lax.rsqrt(jnp.maximum(deg[:, 0], 1.0)), 0.0)
    invc_ad = 1.0 / jnp.maximum(deg[:, 1], 1.0)
    dis_dd = jnp.where(deg[:, 2] > 0, lax.rsqrt(jnp.maximum(deg[:, 2], 1.0)), 0.0)
    invc_ab = 1.0 / jnp.maximum(deg[:, 3], 1.0)
    invc_db = 1.0 / jnp.maximum(deg[:, 4], 1.0)
    return dis_aa, invc_ad, dis_dd, invc_ab, invc_db


def _prep_body(xa_ref, xd_ref, xb_ref, degp_ref,
               Wpa, bpa, Wpd, bpd, Wpb, bpb, tbl_ref, xb_out):
    dis_aa, _, dis_dd, _, _ = _norms_from_deg(degp_ref[...])
    for t in range(TT):
        xa = xa_ref[t] @ Wpa[...].T + bpa[...]
        xd = xd_ref[t] @ Wpd[...].T + bpd[...]
        xb = xb_ref[t] @ Wpb[...].T + bpb[...]
        for h in range(2):
            cs = slice(h * HW, (h + 1) * HW)
            tbl_ref[t, h] = (xa * dis_aa[:, None])[:, cs]
            tbl_ref[4 + t, h] = xa[:, cs]
            tbl_ref[8 + t, h] = (xd * dis_dd[:, None])[:, cs]
            tbl_ref[12 + t, h] = xd[:, cs]
        xb_out[t] = xb


def _tc_prep(x_attk, x_def, x_ball, degp,
             Wp_attk, bp_attk, Wp_def, bp_def, Wp_ball, bp_ball):
    return pl.pallas_call(
        _prep_body,
        grid=(NB,),
        in_specs=[
            pl.BlockSpec((TT, BN, D), lambda i: (0, i, 0)),
            pl.BlockSpec((TT, BN, D), lambda i: (0, i, 0)),
            pl.BlockSpec((TT, BN, D), lambda i: (0, i, 0)),
            pl.BlockSpec((BN, 5, 16), lambda i: (i, 0, 0)),
            pl.BlockSpec((D, D), lambda i: (0, 0)),
            pl.BlockSpec((D,), lambda i: (0,)),
            pl.BlockSpec((D, D), lambda i: (0, 0)),
            pl.BlockSpec((D,), lambda i: (0,)),
            pl.BlockSpec((D, D), lambda i: (0, 0)),
            pl.BlockSpec((D,), lambda i: (0,)),
        ],
        out_specs=[
            pl.BlockSpec((16, 2, BN, HW), lambda i: (0, 0, i, 0)),
            pl.BlockSpec((TT, BN, D), lambda i: (0, i, 0)),
        ],
        out_shape=[
            jax.ShapeDtypeStruct((16, 2, NP, HW), jnp.float32),
            jax.ShapeDtypeStruct((TT, NN, D), jnp.float32),
        ],
    )(x_attk, x_def, x_ball, degp,
      Wp_attk, bp_attk, Wp_def, bp_def, Wp_ball, bp_ball)


def _post_body(agg_ref, tbl_ref, xb_ref, degp_ref,
               Waa, baa, Wdd, bdd,
               Wlad, blad, Wrad, Wlab, blab, Wrab, Wldb, bldb, Wrdb,
               Wih, Whh, bih, bhh, Whp, bhp, out_ref, sums):
    i = pl.program_id(0)

    @pl.when(i == 0)
    def _():
        sums[...] = jnp.zeros_like(sums)

    dis_aa, invc_ad, dis_dd, invc_ab, invc_db = _norms_from_deg(degp_ref[...])

    def cat(ref, j):
        return jnp.concatenate([ref[j, 0], ref[j, 1]], axis=-1)

    za, zd, zb = [], [], []
    for t in range(TT):
        za.append((cat(agg_ref, t) * dis_aa[:, None]) @ Waa[...].T + baa[...])
        zd.append((cat(agg_ref, 4 + t) * invc_ad[:, None]) @ Wlad[...].T + blad[...]
                  + cat(tbl_ref, 12 + t) @ Wrad[...].T
                  + (cat(agg_ref, 8 + t) * dis_dd[:, None]) @ Wdd[...].T + bdd[...])
        zb.append((cat(agg_ref, 12 + t) * invc_ab[:, None]) @ Wlab[...].T + blab[...]
                  + xb_ref[t] @ Wrab[...].T
                  + (cat(agg_ref, 16 + t) * invc_db[:, None]) @ Wldb[...].T + bldb[...]
                  + xb_ref[t] @ Wrdb[...].T)

    hs = []
    for zseq in (za, zd, zb):
        h = jnp.zeros((BN, D), jnp.float32)
        for t in range(TT):
            gi = zseq[t] @ Wih[...].T + bih[...]
            gh = h @ Whh[...].T + bhh[...]
            ir, iz, inn = gi[:, :D], gi[:, D:2 * D], gi[:, 2 * D:]
            hr, hz, hn = gh[:, :D], gh[:, D:2 * D], gh[:, 2 * D:]
            rg = jax.nn.sigmoid(ir + hr)
            zg = jax.nn.sigmoid(iz + hz)
            ng = jnp.tanh(inn + rg * hn)
            h = (1.0 - zg) * ng + zg * h
        hs.append(jnp.sum(h, axis=0))

    sums[...] = sums[...] + jnp.stack(hs, axis=0)

    @pl.when(i == NB - 1)
    def _():
        hcat = (sums[...] / float(NN)).reshape(1, 3 * D)
        out_ref[...] = (hcat @ Whp[...].T + bhp[...][None, :])[0]


def _tc_post(agg, tbl, xb, degp,
             W_gcn_aa, b_gcn_aa, W_gcn_dd, b_gcn_dd,
             Wl_ad, bl_ad, Wr_ad, Wl_ab, bl_ab, Wr_ab, Wl_db, bl_db, Wr_db,
             W_ih, W_hh, b_ih, b_hh, W_hp, b_hp):
    mat = lambda shape: pl.BlockSpec(shape, lambda i: (0,) * len(shape))
    return pl.pallas_call(
        _post_body,
        grid=(NB,),
        in_specs=[
            pl.BlockSpec((20, 2, BN, HW), lambda i: (0, 0, i, 0)),
            pl.BlockSpec((16, 2, BN, HW), lambda i: (0, 0, i, 0)),
            pl.BlockSpec((TT, BN, D), lambda i: (0, i, 0)),
            pl.BlockSpec((BN, 5, 16), lambda i: (i, 0, 0)),
            mat((D, D)), mat((D,)), mat((D, D)), mat((D,)),
            mat((D, D)), mat((D,)), mat((D, D)),
            mat((D, D)), mat((D,)), mat((D, D)),
            mat((D, D)), mat((D,)), mat((D, D)),
            mat((3 * D, D)), mat((3 * D, D)), mat((3 * D,)), mat((3 * D,)),
            mat((D, 3 * D)), mat((D,)),
        ],
        out_specs=pl.BlockSpec((D,), lambda i: (0,)),
        out_shape=jax.ShapeDtypeStruct((D,), jnp.float32),
        scratch_shapes=[pltpu.VMEM((3, D), jnp.float32)],
    )(agg, tbl, xb, degp,
      W_gcn_aa, b_gcn_aa, W_gcn_dd, b_gcn_dd,
      Wl_ad, bl_ad, Wr_ad, Wl_ab, bl_ab, Wr_ab, Wl_db, bl_db, Wr_db,
      W_ih, W_hh, b_ih, b_hh, W_hp, b_hp)


def kernel(x_attk, x_def, x_ball, ei_aa, ei_ad, ei_dd, ei_ab, ei_db,
           Wp_attk, bp_attk, Wp_def, bp_def, Wp_ball, bp_ball,
           W_gcn_aa, b_gcn_aa, W_gcn_dd, b_gcn_dd,
           Wl_ad, bl_ad, Wr_ad, Wl_ab, bl_ab, Wr_ab, Wl_db, bl_db, Wr_db,
           W_ih, W_hh, b_ih, b_hh, W_hp, b_hp):
    srcs, dsts = [], []
    for ei in (ei_aa, ei_ad, ei_dd, ei_ab, ei_db):
        s, d = _pad_edges(ei)
        srcs.append(s)
        dsts.append(d)
    SRC = jnp.stack(srcs)
    DST = jnp.stack(dsts)

    ones_tbl = jnp.ones((1, NP, DEG_W), jnp.float32)
    deg_out = _sc_agg(ones_tbl, DST, DST, 5, DEG_W, "deg")
    degp = jnp.transpose(deg_out, (1, 0, 2))

    tbl, xb = _tc_prep(x_attk, x_def, x_ball, degp,
                       Wp_attk, bp_attk, Wp_def, bp_def, Wp_ball, bp_ball)

    agg = _sc_feat(tbl, SRC, DST)

    return _tc_post(agg, tbl, xb, degp,
                    W_gcn_aa, b_gcn_aa, W_gcn_dd, b_gcn_dd,
                    Wl_ad, bl_ad, Wr_ad, Wl_ab, bl_ab, Wr_ab,
                    Wl_db, bl_db, Wr_db,
                    W_ih, W_hh, b_ih, b_hh, W_hp, b_hp)
